# Initial kernel scaffold; baseline (speedup 1.0000x reference)
import jax
import jax.numpy as jnp
from jax.experimental import pallas as pl


def kernel(x, A, Bw, W, b):
    raise NotImplementedError("write your pallas kernel here")


# speedup vs baseline: 14.1367x; 14.1367x over previous
import jax
import jax.numpy as jnp
from jax.experimental import pallas as pl
from jax.experimental.pallas import tpu as pltpu

K_TOP = 64
SCALE = 2.0
M_TILE = 256
N_SEARCH = 30


def _fused_body(x_ref, at_ref, wt_ref, bwt_ref, b_ref, out_ref):
    x = x_ref[...]
    z = jnp.dot(x, at_ref[...], preferred_element_type=jnp.float32)

    lo = jnp.min(z, axis=1, keepdims=True)
    hi = jnp.max(z, axis=1, keepdims=True)

    def body(_, carry):
        lo, hi = carry
        mid = 0.5 * (lo + hi)
        cnt = jnp.sum((z >= mid).astype(jnp.float32), axis=1, keepdims=True)
        pred = cnt >= float(K_TOP)
        return jnp.where(pred, mid, lo), jnp.where(pred, hi, mid)

    lo, hi = jax.lax.fori_loop(0, N_SEARCH, body, (lo, hi))

    zm = jnp.where(z >= lo, z, 0.0)
    out = jnp.dot(x, wt_ref[...], preferred_element_type=jnp.float32)
    out = out + b_ref[...]
    out = out + SCALE * jnp.dot(zm, bwt_ref[...], preferred_element_type=jnp.float32)
    out_ref[...] = out


def kernel(x, A, Bw, W, b):
    batch, seq, d_in = x.shape
    n = batch * seq
    r = A.shape[0]
    d_out = W.shape[0]
    x2 = x.reshape(n, d_in)

    out = pl.pallas_call(
        _fused_body,
        grid=(n // M_TILE,),
        in_specs=[
            pl.BlockSpec((M_TILE, d_in), lambda i: (i, 0)),
            pl.BlockSpec((d_in, r), lambda i: (0, 0)),
            pl.BlockSpec((d_in, d_out), lambda i: (0, 0)),
            pl.BlockSpec((r, d_out), lambda i: (0, 0)),
            pl.BlockSpec((1, d_out), lambda i: (0, 0)),
        ],
        out_specs=pl.BlockSpec((M_TILE, d_out), lambda i: (i, 0)),
        out_shape=jax.ShapeDtypeStruct((n, d_out), jnp.float32),
        compiler_params=pltpu.CompilerParams(
            dimension_semantics=("parallel",),
        ),
    )(x2, A.T, W.T, Bw.T, b.reshape(1, d_out))
    return out.reshape(batch, seq, d_out)


# speedup vs baseline: 26.8224x; 1.8974x over previous
import jax
import jax.numpy as jnp
from jax.experimental import pallas as pl
from jax.experimental.pallas import tpu as pltpu

K_TOP = 64
SCALE = 2.0
M_TILE = 256
N_SEARCH = 22


def _fused_body(x_ref, a_ref, w_ref, bw_ref, b_ref, out_ref):
    x = x_ref[...]
    xt = x.T
    zt = jnp.dot(a_ref[...], xt, preferred_element_type=jnp.float32)

    lo = jnp.min(zt, axis=0, keepdims=True)
    hi = jnp.max(zt, axis=0, keepdims=True)

    def body(_, carry):
        lo, hi = carry
        mid = 0.5 * (lo + hi)
        cnt = jnp.sum((zt >= mid).astype(jnp.float32), axis=0, keepdims=True)
        pred = cnt >= float(K_TOP)
        return jnp.where(pred, mid, lo), jnp.where(pred, hi, mid)

    lo, hi = jax.lax.fori_loop(0, N_SEARCH, body, (lo, hi))

    zmt = jnp.where(zt >= lo, zt, 0.0)
    ot = jnp.dot(w_ref[...], xt, preferred_element_type=jnp.float32)
    ot = ot + b_ref[...]
    ot = ot + SCALE * jnp.dot(bw_ref[...], zmt, preferred_element_type=jnp.float32)
    out_ref[...] = ot.T


def kernel(x, A, Bw, W, b):
    batch, seq, d_in = x.shape
    n = batch * seq
    r = A.shape[0]
    d_out = W.shape[0]
    x2 = x.reshape(n, d_in)

    out = pl.pallas_call(
        _fused_body,
        grid=(n // M_TILE,),
        in_specs=[
            pl.BlockSpec((M_TILE, d_in), lambda i: (i, 0)),
            pl.BlockSpec((r, d_in), lambda i: (0, 0)),
            pl.BlockSpec((d_out, d_in), lambda i: (0, 0)),
            pl.BlockSpec((d_out, r), lambda i: (0, 0)),
            pl.BlockSpec((d_out, 1), lambda i: (0, 0)),
        ],
        out_specs=pl.BlockSpec((M_TILE, d_out), lambda i: (i, 0)),
        out_shape=jax.ShapeDtypeStruct((n, d_out), jnp.float32),
        compiler_params=pltpu.CompilerParams(
            dimension_semantics=("parallel",),
        ),
    )(x2, A, W, Bw, b.reshape(d_out, 1))
    return out.reshape(batch, seq, d_out)


# speedup vs baseline: 43.2988x; 1.6143x over previous
import jax
import jax.numpy as jnp
from jax.experimental import pallas as pl
from jax.experimental.pallas import tpu as pltpu

K_TOP = 64
SCALE = 2.0
M_TILE = 512
N_SEARCH = 16


def _fused_body(x_ref, a_ref, w_ref, bw_ref, b_ref, out_ref):
    x = x_ref[...]
    xt = x.T
    zt = jnp.dot(a_ref[...], xt, preferred_element_type=jnp.float32)

    lo = jnp.min(zt, axis=0, keepdims=True)
    hi = jnp.max(zt, axis=0, keepdims=True)

    def body(_, carry):
        lo, hi = carry
        mid = 0.5 * (lo + hi)
        cnt = jnp.sum((zt >= mid).astype(jnp.float32), axis=0, keepdims=True)
        pred = cnt >= float(K_TOP)
        return jnp.where(pred, mid, lo), jnp.where(pred, hi, mid)

    lo, hi = jax.lax.fori_loop(0, N_SEARCH, body, (lo, hi))

    zmt = jnp.where(zt >= lo, zt, 0.0)
    ot = jnp.dot(w_ref[...], xt, preferred_element_type=jnp.float32)
    ot = ot + b_ref[...]
    ot = ot + SCALE * jnp.dot(bw_ref[...], zmt, preferred_element_type=jnp.float32)
    out_ref[...] = ot.T


def kernel(x, A, Bw, W, b):
    batch, seq, d_in = x.shape
    n = batch * seq
    r = A.shape[0]
    d_out = W.shape[0]
    x2 = x.reshape(n, d_in)

    out = pl.pallas_call(
        _fused_body,
        grid=(n // M_TILE,),
        in_specs=[
            pl.BlockSpec((M_TILE, d_in), lambda i: (i, 0)),
            pl.BlockSpec((r, d_in), lambda i: (0, 0)),
            pl.BlockSpec((d_out, d_in), lambda i: (0, 0)),
            pl.BlockSpec((d_out, r), lambda i: (0, 0)),
            pl.BlockSpec((d_out, 1), lambda i: (0, 0)),
        ],
        out_specs=pl.BlockSpec((M_TILE, d_out), lambda i: (i, 0)),
        out_shape=jax.ShapeDtypeStruct((n, d_out), jnp.float32),
        compiler_params=pltpu.CompilerParams(
            dimension_semantics=("parallel",),
        ),
    )(x2, A, W, Bw, b.reshape(d_out, 1))
    return out.reshape(batch, seq, d_out)


# speedup vs baseline: 46.5011x; 1.0740x over previous
import jax
import jax.numpy as jnp
from jax.experimental import pallas as pl
from jax.experimental.pallas import tpu as pltpu

K_TOP = 64
SCALE = 2.0
M_TILE = 1024
N_SEARCH = 16


def _fused_body(x_ref, a_ref, w_ref, bw_ref, b_ref, out_ref):
    x = x_ref[...]
    xt = x.T
    zt = jnp.dot(a_ref[...], xt, preferred_element_type=jnp.float32)

    lo = jnp.min(zt, axis=0, keepdims=True)
    hi = jnp.max(zt, axis=0, keepdims=True)

    def body(_, carry):
        lo, hi = carry
        mid = 0.5 * (lo + hi)
        cnt = jnp.sum((zt >= mid).astype(jnp.float32), axis=0, keepdims=True)
        pred = cnt >= float(K_TOP)
        return jnp.where(pred, mid, lo), jnp.where(pred, hi, mid)

    lo, hi = jax.lax.fori_loop(0, N_SEARCH, body, (lo, hi))

    zmt = jnp.where(zt >= lo, zt, 0.0)
    ot = jnp.dot(w_ref[...], xt, preferred_element_type=jnp.float32)
    ot = ot + b_ref[...]
    ot = ot + SCALE * jnp.dot(bw_ref[...], zmt, preferred_element_type=jnp.float32)
    out_ref[...] = ot.T


def kernel(x, A, Bw, W, b):
    batch, seq, d_in = x.shape
    n = batch * seq
    r = A.shape[0]
    d_out = W.shape[0]
    x2 = x.reshape(n, d_in)

    out = pl.pallas_call(
        _fused_body,
        grid=(n // M_TILE,),
        in_specs=[
            pl.BlockSpec((M_TILE, d_in), lambda i: (i, 0)),
            pl.BlockSpec((r, d_in), lambda i: (0, 0)),
            pl.BlockSpec((d_out, d_in), lambda i: (0, 0)),
            pl.BlockSpec((d_out, r), lambda i: (0, 0)),
            pl.BlockSpec((d_out, 1), lambda i: (0, 0)),
        ],
        out_specs=pl.BlockSpec((M_TILE, d_out), lambda i: (i, 0)),
        out_shape=jax.ShapeDtypeStruct((n, d_out), jnp.float32),
        compiler_params=pltpu.CompilerParams(
            dimension_semantics=("parallel",),
        ),
    )(x2, A, W, Bw, b.reshape(d_out, 1))
    return out.reshape(batch, seq, d_out)
